# baseline (speedup 1.0000x reference)
import jax
import jax.numpy as jnp
from jax import lax
from jax.experimental import pallas as pl
from jax.experimental.pallas import tpu as pltpu
from jax.experimental.pallas import tpu_sc as plsc

_B = 2
_H = _W = _D = 128
_NVOX = _B * _H * _W * _D
_NW = 32
_VT = _NVOX // _NW
_C = 2048
_NCH = _VT // _C
_G = _C // 16
_NS = _C // 128


def _sc_body(w8, flow, out, fbuf, ibuf, wbuf, mbuf, gbuf, obuf, gsem):
    cid = lax.axis_index("c")
    sid = lax.axis_index("s")
    wid = sid * 2 + cid
    vbase = wid * _VT

    iota = lax.iota(jnp.int32, 16)
    iota3 = iota * 3
    iota_f = iota.astype(jnp.float32)

    def chunk(ch, carry):
        v0 = vbase + ch * _C
        pltpu.sync_copy(flow.at[pl.ds(v0 * 3, _C * 3)], fbuf)
        batch = v0 // (_H * _W * _D)
        bb = batch * (_H * _W * _D)
        yy = (v0 // (_W * _D)) % _H
        xb = (v0 // _D) % _W
        y_f = yy.astype(jnp.float32)

        def grp(g, c2):
            o = g * 16
            fo = o * 3
            dyv = plsc.load_gather(fbuf, [fo + iota3])
            dxv = plsc.load_gather(fbuf, [fo + iota3 + 1])
            dzv = plsc.load_gather(fbuf, [fo + iota3 + 2])
            x_f = (xb + g // 8).astype(jnp.float32)
            zb_f = ((g % 8) * 16).astype(jnp.float32)
            xn = dxv + x_f
            yn = dyv + y_f
            zn = dzv + (zb_f + iota_f)

            def ffloor(v):
                t = v.astype(jnp.int32)
                tf = t.astype(jnp.float32)
                i0 = jnp.where(tf > v, t - 1, t)
                return i0, v - i0.astype(jnp.float32)

            x0, fx = ffloor(xn)
            y0, fy = ffloor(yn)
            z0, fz = ffloor(zn)
            x0c = jnp.clip(x0, 0, _W - 1)
            x1c = jnp.clip(x0 + 1, 0, _W - 1)
            y0c = jnp.clip(y0, 0, _H - 1)
            y1c = jnp.clip(y0 + 1, 0, _H - 1)
            s = jnp.clip(z0, 0, _D - 2)
            edge = (z0 < 0) | (z0 >= _D - 1)
            fzc = jnp.where(edge, jnp.where(z0 >= _D - 1, 1.0, 0.0), fz)
            gx = 1.0 - fx
            gy = 1.0 - fy
            ry0 = y0c * (_W * _D) + (bb + s)
            ry1 = y1c * (_W * _D) + (bb + s)
            rx0 = x0c * _D
            rx1 = x1c * _D
            ghi = g // 8
            olo = (g % 8) * 16
            ibuf[0, ghi, pl.ds(olo, 16)] = (ry0 + rx0) >> 2
            ibuf[1, ghi, pl.ds(olo, 16)] = (ry0 + rx1) >> 2
            ibuf[2, ghi, pl.ds(olo, 16)] = (ry1 + rx0) >> 2
            ibuf[3, ghi, pl.ds(olo, 16)] = (ry1 + rx1) >> 2
            wbuf[0, pl.ds(o, 16)] = gx * gy
            wbuf[1, pl.ds(o, 16)] = fx * gy
            wbuf[2, pl.ds(o, 16)] = gx * fy
            wbuf[3, pl.ds(o, 16)] = fx * fy
            wbuf[4, pl.ds(o, 16)] = fzc
            mbuf[pl.ds(o, 16)] = s & 3
            return c2

        lax.fori_loop(0, _G, grp, 0)

        def fire(j, c2):
            cps = [
                pltpu.async_copy(
                    w8.at[ibuf.at[c4, j]],
                    gbuf.at[c4, pl.ds(j * 128, 128)],
                    gsem,
                )
                for c4 in range(4)
            ]
            for cp in cps:
                cp.wait()
            return c2

        lax.fori_loop(0, _NS, fire, 0)

        def blend(g, c2):
            o = g * 16
            fzc = wbuf[4, pl.ds(o, 16)]
            m = mbuf[pl.ds(o, 16)]
            acc = iota_f * 0.0
            for c4 in range(4):
                vz0 = plsc.load_gather(gbuf.at[c4], [o + iota, m])
                vz1 = plsc.load_gather(gbuf.at[c4], [o + iota, m + 1])
                wc = wbuf[c4, pl.ds(o, 16)]
                acc = acc + wc * (vz0 + fzc * (vz1 - vz0))
            obuf[pl.ds(o, 16)] = acc
            return c2

        lax.fori_loop(0, _G, blend, 0)
        pltpu.sync_copy(obuf, out.at[pl.ds(v0, _C)])
        return carry

    lax.fori_loop(0, _NCH, chunk, 0)


def kernel(I, flow):
    flat = I.reshape(_NVOX)
    padded = jnp.concatenate([flat, jnp.zeros((8,), jnp.float32)])
    w8 = jnp.stack(
        [lax.slice(padded, (j,), (j + _NVOX,), (4,)) for j in range(8)], axis=-1
    )
    flow_flat = flow.reshape(_NVOX * 3)

    mesh = plsc.VectorSubcoreMesh(
        core_axis_name="c", subcore_axis_name="s", num_cores=2, num_subcores=16
    )
    f = pl.kernel(
        _sc_body,
        out_type=jax.ShapeDtypeStruct((_NVOX,), jnp.float32),
        mesh=mesh,
        compiler_params=pltpu.CompilerParams(
            needs_layout_passes=False, use_tc_tiling_on_sc=False
        ),
        scratch_types=[
            pltpu.VMEM((_C * 3,), jnp.float32),
            pltpu.VMEM((4, _NS, 128), jnp.int32),
            pltpu.VMEM((5, _C), jnp.float32),
            pltpu.VMEM((_C,), jnp.int32),
            pltpu.VMEM((4, _C, 8), jnp.float32),
            pltpu.VMEM((_C,), jnp.float32),
            pltpu.SemaphoreType.DMA,
        ],
    )
    outf = f(w8, flow_flat)
    return outf.reshape(_B, _H, _W, _D, 1)


# speedup vs baseline: 1.1199x; 1.1199x over previous
import jax
import jax.numpy as jnp
from jax import lax
from jax.experimental import pallas as pl
from jax.experimental.pallas import tpu as pltpu
from jax.experimental.pallas import tpu_sc as plsc

_B = 2
_H = _W = _D = 128
_NVOX = _B * _H * _W * _D
_NW = 32
_VT = _NVOX // _NW
_C = 2048
_NCH = _VT // _C
_G = _C // 16
_NS = _C // 128


def _sc_body(w8, flow, out, fbuf, ibuf, wbuf, mbuf, gbuf, obuf, gsem):
    cid = lax.axis_index("c")
    sid = lax.axis_index("s")
    wid = sid * 2 + cid
    vbase = wid * _VT

    iota = lax.iota(jnp.int32, 16)
    iota3 = iota * 3
    iota_f = iota.astype(jnp.float32)

    def chunk(ch, carry):
        v0 = vbase + ch * _C
        pltpu.sync_copy(flow.at[pl.ds(v0 * 3, _C * 3)], fbuf)
        batch = v0 // (_H * _W * _D)
        bb = batch * (_H * _W * _D)
        yy = (v0 // (_W * _D)) % _H
        xb = (v0 // _D) % _W
        y_f = yy.astype(jnp.float32)

        def grp(g, c2):
            o = g * 16
            fo = o * 3
            dyv = plsc.load_gather(fbuf, [fo + iota3])
            dxv = plsc.load_gather(fbuf, [fo + iota3 + 1])
            dzv = plsc.load_gather(fbuf, [fo + iota3 + 2])
            x_f = (xb + g // 8).astype(jnp.float32)
            zb_f = ((g % 8) * 16).astype(jnp.float32)
            xn = dxv + x_f
            yn = dyv + y_f
            zn = dzv + (zb_f + iota_f)

            def ffloor(v):
                t = v.astype(jnp.int32)
                tf = t.astype(jnp.float32)
                i0 = jnp.where(tf > v, t - 1, t)
                return i0, v - i0.astype(jnp.float32)

            x0, fx = ffloor(xn)
            y0, fy = ffloor(yn)
            z0, fz = ffloor(zn)
            x0c = jnp.clip(x0, 0, _W - 1)
            x1c = jnp.clip(x0 + 1, 0, _W - 1)
            y0c = jnp.clip(y0, 0, _H - 1)
            y1c = jnp.clip(y0 + 1, 0, _H - 1)
            s = jnp.clip(z0, 0, _D - 2)
            edge = (z0 < 0) | (z0 >= _D - 1)
            fzc = jnp.where(edge, jnp.where(z0 >= _D - 1, 1.0, 0.0), fz)
            gx = 1.0 - fx
            gy = 1.0 - fy
            ry0 = y0c * (_W * _D) + (bb + s)
            ry1 = y1c * (_W * _D) + (bb + s)
            rx0 = x0c * _D
            rx1 = x1c * _D
            ghi = g // 8
            olo = (g % 8) * 16
            ibuf[0, ghi, pl.ds(olo, 16)] = (ry0 + rx0) >> 2
            ibuf[1, ghi, pl.ds(olo, 16)] = (ry0 + rx1) >> 2
            ibuf[2, ghi, pl.ds(olo, 16)] = (ry1 + rx0) >> 2
            ibuf[3, ghi, pl.ds(olo, 16)] = (ry1 + rx1) >> 2
            wbuf[0, pl.ds(o, 16)] = gx * gy
            wbuf[1, pl.ds(o, 16)] = fx * gy
            wbuf[2, pl.ds(o, 16)] = gx * fy
            wbuf[3, pl.ds(o, 16)] = fx * fy
            wbuf[4, pl.ds(o, 16)] = fzc
            mbuf[pl.ds(o, 16)] = s & 3
            return c2

        lax.fori_loop(0, _G, grp, 0)

        def fire(j, c2):
            cps = [
                pltpu.async_copy(
                    w8.at[ibuf.at[c4, j]],
                    gbuf.at[c4, pl.ds(j * 128, 128)],
                    gsem,
                )
                for c4 in range(4)
            ]
            for cp in cps:
                cp.wait()
            return c2

        lax.fori_loop(0, _NS, fire, 0)

        def blend(g, c2):
            o = g * 16
            fzc = wbuf[4, pl.ds(o, 16)]
            m = mbuf[pl.ds(o, 16)]
            acc = iota_f * 0.0
            for c4 in range(4):
                vz0 = plsc.load_gather(gbuf.at[c4], [o + iota, m])
                vz1 = plsc.load_gather(gbuf.at[c4], [o + iota, m + 1])
                wc = wbuf[c4, pl.ds(o, 16)]
                acc = acc + wc * (vz0 + fzc * (vz1 - vz0))
            obuf[pl.ds(o, 16)] = acc
            return c2

        lax.fori_loop(0, _G, blend, 0)
        pltpu.sync_copy(obuf, out.at[pl.ds(v0, _C)])
        return carry

    lax.fori_loop(0, _NCH, chunk, 0)


def kernel(I, flow):
    flat = I.reshape(_NVOX)
    a = flat.reshape(_NVOX // 4, 4)
    a_next = jnp.concatenate([a[1:], jnp.zeros((1, 4), jnp.float32)], axis=0)
    w8 = jnp.concatenate([a, a_next], axis=1)
    flow_flat = flow.reshape(_NVOX * 3)

    mesh = plsc.VectorSubcoreMesh(
        core_axis_name="c", subcore_axis_name="s", num_cores=2, num_subcores=16
    )
    f = pl.kernel(
        _sc_body,
        out_type=jax.ShapeDtypeStruct((_NVOX,), jnp.float32),
        mesh=mesh,
        compiler_params=pltpu.CompilerParams(
            needs_layout_passes=False, use_tc_tiling_on_sc=False
        ),
        scratch_types=[
            pltpu.VMEM((_C * 3,), jnp.float32),
            pltpu.VMEM((4, _NS, 128), jnp.int32),
            pltpu.VMEM((5, _C), jnp.float32),
            pltpu.VMEM((_C,), jnp.int32),
            pltpu.VMEM((4, _C, 8), jnp.float32),
            pltpu.VMEM((_C,), jnp.float32),
            pltpu.SemaphoreType.DMA,
        ],
    )
    outf = f(w8, flow_flat)
    return outf.reshape(_B, _H, _W, _D, 1)


# speedup vs baseline: 1.2453x; 1.1120x over previous
import jax
import jax.numpy as jnp
from jax import lax
from jax.experimental import pallas as pl
from jax.experimental.pallas import tpu as pltpu
from jax.experimental.pallas import tpu_sc as plsc

_B = 2
_H = _W = _D = 128
_NVOX = _B * _H * _W * _D
_NW = 32
_VT = _NVOX // _NW
_C = 2048
_NCH = _VT // _C
_G = _C // 16
_NS = _C // 128


def _sc_body(w8, flow, out, fbuf, ibuf, wbuf, mbuf, gbuf, obuf, gsem):
    cid = lax.axis_index("c")
    sid = lax.axis_index("s")
    wid = sid * 2 + cid
    vbase = wid * _VT

    iota = lax.iota(jnp.int32, 16)
    iota3 = iota * 3
    iota_f = iota.astype(jnp.float32)

    def chunk(ch, carry):
        v0 = vbase + ch * _C
        pltpu.sync_copy(flow.at[pl.ds(v0 * 3, _C * 3)], fbuf)
        batch = v0 // (_H * _W * _D)
        bb = batch * (_H * _W * _D)
        yy = (v0 // (_W * _D)) % _H
        xb = (v0 // _D) % _W
        y_f = yy.astype(jnp.float32)

        def grp(g, c2):
            o = g * 16
            fo = o * 3
            dyv = plsc.load_gather(fbuf, [fo + iota3])
            dxv = plsc.load_gather(fbuf, [fo + iota3 + 1])
            dzv = plsc.load_gather(fbuf, [fo + iota3 + 2])
            x_f = (xb + g // 8).astype(jnp.float32)
            zb_f = ((g % 8) * 16).astype(jnp.float32)
            xn = dxv + x_f
            yn = dyv + y_f
            zn = dzv + (zb_f + iota_f)

            def ffloor(v):
                t = v.astype(jnp.int32)
                tf = t.astype(jnp.float32)
                i0 = jnp.where(tf > v, t - 1, t)
                return i0, v - i0.astype(jnp.float32)

            x0, fx = ffloor(xn)
            y0, fy = ffloor(yn)
            z0, fz = ffloor(zn)
            x0c = jnp.clip(x0, 0, _W - 1)
            x1c = jnp.clip(x0 + 1, 0, _W - 1)
            y0c = jnp.clip(y0, 0, _H - 1)
            y1c = jnp.clip(y0 + 1, 0, _H - 1)
            s = jnp.clip(z0, 0, _D - 2)
            edge = (z0 < 0) | (z0 >= _D - 1)
            fzc = jnp.where(edge, jnp.where(z0 >= _D - 1, 1.0, 0.0), fz)
            gx = 1.0 - fx
            gy = 1.0 - fy
            ry0 = y0c * (_W * _D) + (bb + s)
            ry1 = y1c * (_W * _D) + (bb + s)
            rx0 = x0c * _D
            rx1 = x1c * _D
            ghi = g // 8
            olo = (g % 8) * 16
            ibuf[0, ghi, pl.ds(olo, 16)] = (ry0 + rx0) >> 2
            ibuf[1, ghi, pl.ds(olo, 16)] = (ry0 + rx1) >> 2
            ibuf[2, ghi, pl.ds(olo, 16)] = (ry1 + rx0) >> 2
            ibuf[3, ghi, pl.ds(olo, 16)] = (ry1 + rx1) >> 2
            wbuf[0, pl.ds(o, 16)] = gx * gy
            wbuf[1, pl.ds(o, 16)] = fx * gy
            wbuf[2, pl.ds(o, 16)] = gx * fy
            wbuf[3, pl.ds(o, 16)] = fx * fy
            wbuf[4, pl.ds(o, 16)] = fzc
            mbuf[pl.ds(o, 16)] = s & 3
            return c2

        lax.fori_loop(0, _G, grp, 0)

        def fire(j, c2):
            cps = [
                pltpu.async_copy(
                    w8.at[ibuf.at[c4, j]],
                    gbuf.at[c4, pl.ds(j * 128, 128)],
                    gsem,
                )
                for c4 in range(4)
            ]
            for cp in cps:
                cp.wait()
            return c2

        lax.fori_loop(0, _NS, fire, 0)

        def blend(g, c2):
            o = g * 16
            fzc = wbuf[4, pl.ds(o, 16)]
            m = mbuf[pl.ds(o, 16)]
            acc = iota_f * 0.0
            for c4 in range(4):
                vz0 = plsc.load_gather(gbuf.at[c4], [o + iota, m])
                vz1 = plsc.load_gather(gbuf.at[c4], [o + iota, m + 1])
                wc = wbuf[c4, pl.ds(o, 16)]
                acc = acc + wc * (vz0 + fzc * (vz1 - vz0))
            obuf[pl.ds(o, 16)] = acc
            return c2

        lax.fori_loop(0, _G, blend, 0)
        pltpu.sync_copy(obuf, out.at[pl.ds(v0, _C)])
        return carry

    lax.fori_loop(0, _NCH, chunk, 0)


_N4 = _NVOX // 4
_RP = 4096


_MROWS = _NVOX // 128
_RP2 = 1024


def _prep_body(in1, in2, out_ref):
    i_id = lax.broadcasted_iota(jnp.int32, (128, 128), 0)
    l_id = lax.broadcasted_iota(jnp.int32, (128, 128), 1)
    perm = 4 * (l_id >> 3) + (l_id & 7)
    p0 = (i_id == perm).astype(jnp.float32)
    p1 = (i_id == perm + 64).astype(jnp.float32)
    p2 = (i_id == perm - 64).astype(jnp.float32)
    x = in1[...]
    rolled = jnp.concatenate([x[1:], in2[0:1]], axis=0)
    hi = lax.Precision.HIGHEST
    out_ref[:, 0, :] = jnp.dot(x, p0, precision=hi, preferred_element_type=jnp.float32)
    out_ref[:, 1, :] = jnp.dot(
        x, p1, precision=hi, preferred_element_type=jnp.float32
    ) + jnp.dot(rolled, p2, precision=hi, preferred_element_type=jnp.float32)


def _build_w8(flat):
    m = flat.reshape(_MROWS, 128)
    ng = _MROWS // _RP2
    out3 = pl.pallas_call(
        _prep_body,
        grid=(ng,),
        in_specs=[
            pl.BlockSpec((_RP2, 128), lambda g: (g, 0)),
            pl.BlockSpec((_RP2, 128), lambda g: (jnp.minimum(g + 1, ng - 1), 0)),
        ],
        out_specs=pl.BlockSpec((_RP2, 2, 128), lambda g: (g, 0, 0)),
        out_shape=jax.ShapeDtypeStruct((_MROWS, 2, 128), jnp.float32),
    )(m, m)
    return out3.reshape(_N4, 8)


def kernel(I, flow):
    flat = I.reshape(_NVOX)
    w8 = _build_w8(flat)
    flow_flat = flow.reshape(_NVOX * 3)

    mesh = plsc.VectorSubcoreMesh(
        core_axis_name="c", subcore_axis_name="s", num_cores=2, num_subcores=16
    )
    f = pl.kernel(
        _sc_body,
        out_type=jax.ShapeDtypeStruct((_NVOX,), jnp.float32),
        mesh=mesh,
        compiler_params=pltpu.CompilerParams(
            needs_layout_passes=False, use_tc_tiling_on_sc=False
        ),
        scratch_types=[
            pltpu.VMEM((_C * 3,), jnp.float32),
            pltpu.VMEM((4, _NS, 128), jnp.int32),
            pltpu.VMEM((5, _C), jnp.float32),
            pltpu.VMEM((_C,), jnp.int32),
            pltpu.VMEM((4, _C, 8), jnp.float32),
            pltpu.VMEM((_C,), jnp.float32),
            pltpu.SemaphoreType.DMA,
        ],
    )
    outf = f(w8, flow_flat)
    return outf.reshape(_B, _H, _W, _D, 1)
